# baseline (speedup 1.0000x reference)
import functools

import jax
import jax.numpy as jnp
from jax import lax
from jax.experimental import pallas as pl
from jax.experimental.pallas import tpu as pltpu
from jax.experimental.pallas import tpu_sc as plsc

N = 4096
D = 512
K = 2048
_NH = N // 2

_CHUNKS = (768, 768, 512)

_NC, _NS = 2, 16
_NW = _NC * _NS
_CH = 8
_L = 16
_MASK = jnp.uint32(0xFFFF0000)


def _pack_rows(src, dst):
    def body(j, _):
        col = j * _L
        for r in range(_CH):
            lo = lax.bitcast_convert_type(src[r, pl.ds(col, _L)], jnp.uint32)
            hi = lax.bitcast_convert_type(src[r, pl.ds(_NH + col, _L)],
                                          jnp.uint32)
            w = (lo >> 16) | (hi & _MASK)
            dst[r, pl.ds(col, _L)] = lax.bitcast_convert_type(w, jnp.int32)
        return 0
    lax.fori_loop(0, _NH // _L, body, 0)


def _gather_body(a_hbm, idx_hbm, out_hbm, idx_v, buf0, buf1, ob0, ob1,
                 gsem0, gsem1, wsem0, wsem1, *, chunk_off, bpw, nsub):
    wid = lax.axis_index("s") * _NC + lax.axis_index("c")
    base = wid * bpw
    pltpu.sync_copy(idx_hbm.at[pl.ds(chunk_off + base, bpw)], idx_v)
    bufs = (buf0, buf1)
    obufs = (ob0, ob1)
    gsems = (gsem0, gsem1)
    wsems = (wsem0, wsem1)
    gpend = [None, None]
    wpend = [None, None]
    gpend[0] = pltpu.async_copy(
        a_hbm.at[idx_v.at[pl.ds(0, _CH)]], bufs[0], gsems[0])
    for c in range(nsub):
        slot = c % 2
        nxt = (c + 1) % 2
        gpend[slot].wait()
        if c + 1 < nsub:
            gpend[nxt] = pltpu.async_copy(
                a_hbm.at[idx_v.at[pl.ds((c + 1) * _CH, _CH)]],
                bufs[nxt], gsems[nxt])
        if wpend[slot] is not None:
            wpend[slot].wait()
        _pack_rows(bufs[slot], obufs[slot])
        wpend[slot] = pltpu.async_copy(
            obufs[slot], out_hbm.at[pl.ds(base + c * _CH, _CH)], wsems[slot])
    for p in wpend:
        if p is not None:
            p.wait()


def _gather_rows(a, idx, chunk_off, kc):
    bpw = kc // _NW
    mesh = plsc.VectorSubcoreMesh(core_axis_name="c", subcore_axis_name="s")
    return pl.kernel(
        functools.partial(_gather_body, chunk_off=chunk_off, bpw=bpw,
                          nsub=bpw // _CH),
        mesh=mesh,
        out_type=jax.ShapeDtypeStruct((kc, _NH), jnp.int32),
        scratch_types=[
            pltpu.VMEM((bpw,), jnp.int32),
            pltpu.VMEM((_CH, N), jnp.float32),
            pltpu.VMEM((_CH, N), jnp.float32),
            pltpu.VMEM((_CH, _NH), jnp.int32),
            pltpu.VMEM((_CH, _NH), jnp.int32),
            pltpu.SemaphoreType.DMA,
            pltpu.SemaphoreType.DMA,
            pltpu.SemaphoreType.DMA,
            pltpu.SemaphoreType.DMA,
        ],
    )(a, idx)


_BX = 1024


def _xw_body(x_ref, wt_ref, out_ref):
    out_ref[...] = jnp.dot(
        x_ref[...].astype(jnp.bfloat16), wt_ref[...].astype(jnp.bfloat16),
        preferred_element_type=jnp.float32).astype(jnp.bfloat16)


def _xw(x, wt):
    return pl.pallas_call(
        _xw_body,
        grid=(N // _BX,),
        in_specs=[
            pl.BlockSpec((_BX, D), lambda i: (i, 0)),
            pl.BlockSpec((D, D), lambda i: (0, 0)),
        ],
        out_specs=pl.BlockSpec((_BX, D), lambda i: (i, 0)),
        out_shape=jax.ShapeDtypeStruct((N, D), jnp.bfloat16),
        compiler_params=pltpu.CompilerParams(
            dimension_semantics=("arbitrary",)),
    )(x, wt)


_BM = 256


def _mm_body(ag_ref, xwt_ref, xwb_ref, b_ref, val_ref, *rest):
    out_ref = rest[-1]
    v = ag_ref[...]
    fa = lax.bitcast_convert_type(v << 16, jnp.float32)
    fb = lax.bitcast_convert_type(v & jnp.int32(-65536), jnp.float32)
    h = jnp.dot(fa.astype(jnp.bfloat16), xwt_ref[...],
                preferred_element_type=jnp.float32)
    h += jnp.dot(fb.astype(jnp.bfloat16), xwb_ref[...],
                 preferred_element_type=jnp.float32)
    out_ref[...] = (h + b_ref[...]) * val_ref[...]


def _matmul_chunk(row_off, kc, ag, xw_bf, b2d, val2d, prev):
    off = row_off // _BM
    in_specs = [
        pl.BlockSpec((_BM, _NH), lambda i: (i, 0)),
        pl.BlockSpec((_NH, D), lambda i: (0, 0)),
        pl.BlockSpec((_NH, D), lambda i: (1, 0)),
        pl.BlockSpec((1, D), lambda i: (0, 0)),
        pl.BlockSpec((_BM, 1), lambda i: (off + i, 0)),
    ]
    args = [ag, xw_bf, xw_bf, b2d, val2d]
    aliases = {}
    if prev is not None:
        in_specs.append(pl.BlockSpec(memory_space=pl.ANY))
        args.append(prev)
        aliases = {5: 0}
    return pl.pallas_call(
        _mm_body,
        grid=(kc // _BM,),
        in_specs=in_specs,
        out_specs=pl.BlockSpec((_BM, D), lambda i: (off + i, 0)),
        out_shape=jax.ShapeDtypeStruct((K, D), jnp.float32),
        input_output_aliases=aliases,
        compiler_params=pltpu.CompilerParams(
            dimension_semantics=("arbitrary",)),
    )(*args)


def kernel(A, X, idx, value, W, b):
    idx32 = idx.astype(jnp.int32)
    b2d = b.reshape(1, D)
    val2d = value.reshape(K, 1)
    offs = [0]
    for kc in _CHUNKS[:-1]:
        offs.append(offs[-1] + kc)
    ags = [_gather_rows(A, idx32, off, kc)
           for off, kc in zip(offs, _CHUNKS)]
    xw_bf = _xw(X, W.T)
    out = None
    for off, kc, ag in zip(offs, _CHUNKS, ags):
        out = _matmul_chunk(off, kc, ag, xw_bf, b2d, val2d, out)
    return out


# speedup vs baseline: 1.0840x; 1.0840x over previous
import functools

import jax
import jax.numpy as jnp
from jax import lax
from jax.experimental import pallas as pl
from jax.experimental.pallas import tpu as pltpu
from jax.experimental.pallas import tpu_sc as plsc

N = 4096
D = 512
K = 2048
_NH = N // 2

_CHUNKS = (1280, 768)

_NC, _NS = 2, 16
_NW = _NC * _NS
_CH = 8
_L = 16
_MASK = jnp.uint32(0xFFFF0000)


def _pack_rows(src, dst):
    def body(j, _):
        col = j * _L
        for r in range(_CH):
            lo = lax.bitcast_convert_type(src[r, pl.ds(col, _L)], jnp.uint32)
            hi = lax.bitcast_convert_type(src[r, pl.ds(_NH + col, _L)],
                                          jnp.uint32)
            w = (lo >> 16) | (hi & _MASK)
            dst[r, pl.ds(col, _L)] = lax.bitcast_convert_type(w, jnp.int32)
        return 0
    lax.fori_loop(0, _NH // _L, body, 0)


def _gather_body(a_hbm, idx_hbm, out_hbm, idx_v, buf0, buf1, ob0, ob1,
                 gsem0, gsem1, wsem0, wsem1, *, chunk_off, bpw, nsub):
    wid = lax.axis_index("s") * _NC + lax.axis_index("c")
    base = wid * bpw
    pltpu.sync_copy(idx_hbm.at[pl.ds(chunk_off + base, bpw)], idx_v)
    bufs = (buf0, buf1)
    obufs = (ob0, ob1)
    gsems = (gsem0, gsem1)
    wsems = (wsem0, wsem1)
    gpend = [None, None]
    wpend = [None, None]
    gpend[0] = pltpu.async_copy(
        a_hbm.at[idx_v.at[pl.ds(0, _CH)]], bufs[0], gsems[0])
    for c in range(nsub):
        slot = c % 2
        nxt = (c + 1) % 2
        gpend[slot].wait()
        if c + 1 < nsub:
            gpend[nxt] = pltpu.async_copy(
                a_hbm.at[idx_v.at[pl.ds((c + 1) * _CH, _CH)]],
                bufs[nxt], gsems[nxt])
        if wpend[slot] is not None:
            wpend[slot].wait()
        _pack_rows(bufs[slot], obufs[slot])
        wpend[slot] = pltpu.async_copy(
            obufs[slot], out_hbm.at[pl.ds(base + c * _CH, _CH)], wsems[slot])
    for p in wpend:
        if p is not None:
            p.wait()


def _gather_rows(a, idx, chunk_off, kc):
    bpw = kc // _NW
    mesh = plsc.VectorSubcoreMesh(core_axis_name="c", subcore_axis_name="s")
    return pl.kernel(
        functools.partial(_gather_body, chunk_off=chunk_off, bpw=bpw,
                          nsub=bpw // _CH),
        mesh=mesh,
        out_type=jax.ShapeDtypeStruct((kc, _NH), jnp.int32),
        scratch_types=[
            pltpu.VMEM((bpw,), jnp.int32),
            pltpu.VMEM((_CH, N), jnp.float32),
            pltpu.VMEM((_CH, N), jnp.float32),
            pltpu.VMEM((_CH, _NH), jnp.int32),
            pltpu.VMEM((_CH, _NH), jnp.int32),
            pltpu.SemaphoreType.DMA,
            pltpu.SemaphoreType.DMA,
            pltpu.SemaphoreType.DMA,
            pltpu.SemaphoreType.DMA,
        ],
    )(a, idx)


_BX = 1024


def _xw_body(x_ref, wt_ref, out_ref):
    out_ref[...] = jnp.dot(
        x_ref[...].astype(jnp.bfloat16), wt_ref[...].astype(jnp.bfloat16),
        preferred_element_type=jnp.float32).astype(jnp.bfloat16)


def _xw(x, wt):
    return pl.pallas_call(
        _xw_body,
        grid=(N // _BX,),
        in_specs=[
            pl.BlockSpec((_BX, D), lambda i: (i, 0)),
            pl.BlockSpec((D, D), lambda i: (0, 0)),
        ],
        out_specs=pl.BlockSpec((_BX, D), lambda i: (i, 0)),
        out_shape=jax.ShapeDtypeStruct((N, D), jnp.bfloat16),
        compiler_params=pltpu.CompilerParams(
            dimension_semantics=("arbitrary",)),
    )(x, wt)


_BM = 256


def _mm_body(ag_ref, xwt_ref, xwb_ref, b_ref, val_ref, *rest):
    out_ref = rest[-1]
    v = ag_ref[...]
    fa = lax.bitcast_convert_type(v << 16, jnp.float32)
    fb = lax.bitcast_convert_type(v & jnp.int32(-65536), jnp.float32)
    h = jnp.dot(fa.astype(jnp.bfloat16), xwt_ref[...],
                preferred_element_type=jnp.float32)
    h += jnp.dot(fb.astype(jnp.bfloat16), xwb_ref[...],
                 preferred_element_type=jnp.float32)
    out_ref[...] = (h + b_ref[...]) * val_ref[...]


def _matmul_chunk(row_off, kc, ag, xw_bf, b2d, val2d, prev):
    off = row_off // _BM
    in_specs = [
        pl.BlockSpec((_BM, _NH), lambda i: (i, 0)),
        pl.BlockSpec((_NH, D), lambda i: (0, 0)),
        pl.BlockSpec((_NH, D), lambda i: (1, 0)),
        pl.BlockSpec((1, D), lambda i: (0, 0)),
        pl.BlockSpec((_BM, 1), lambda i: (off + i, 0)),
    ]
    args = [ag, xw_bf, xw_bf, b2d, val2d]
    aliases = {}
    if prev is not None:
        in_specs.append(pl.BlockSpec(memory_space=pl.ANY))
        args.append(prev)
        aliases = {5: 0}
    return pl.pallas_call(
        _mm_body,
        grid=(kc // _BM,),
        in_specs=in_specs,
        out_specs=pl.BlockSpec((_BM, D), lambda i: (off + i, 0)),
        out_shape=jax.ShapeDtypeStruct((K, D), jnp.float32),
        input_output_aliases=aliases,
        compiler_params=pltpu.CompilerParams(
            dimension_semantics=("arbitrary",)),
    )(*args)


def kernel(A, X, idx, value, W, b):
    idx32 = idx.astype(jnp.int32)
    b2d = b.reshape(1, D)
    val2d = value.reshape(K, 1)
    offs = [0]
    for kc in _CHUNKS[:-1]:
        offs.append(offs[-1] + kc)
    ags = [_gather_rows(A, idx32, off, kc)
           for off, kc in zip(offs, _CHUNKS)]
    xw_bf = _xw(X, W.T)
    out = None
    for off, kc, ag in zip(offs, _CHUNKS, ags):
        out = _matmul_chunk(off, kc, ag, xw_bf, b2d, val2d, out)
    return out
